# Initial kernel scaffold; baseline (speedup 1.0000x reference)
import jax
import jax.numpy as jnp
from jax.experimental import pallas as pl


def kernel(x, edge_index, W1, b1, W2, b2):
    raise NotImplementedError("write your pallas kernel here")


# speedup vs baseline: 13.2435x; 13.2435x over previous
import functools

import jax
import jax.numpy as jnp
from jax import lax
from jax.experimental import pallas as pl
from jax.experimental.pallas import tpu as pltpu
from jax.experimental.pallas import tpu_sc as plsc

N = 10000
NP = 10240
E = 320000
NC, NS = 2, 16
NW = NC * NS
EPW = E // NW
CH = 80
NITER = EPW // CH
RPT = NP // NS
R = 1000
GRID = N // R

_MESH = dict(core_axis_name="c", subcore_axis_name="s", num_cores=NC,
             num_subcores=NS)


@functools.partial(
    pl.kernel,
    out_type=jax.ShapeDtypeStruct((NC, NP), jnp.float32),
    mesh=plsc.VectorSubcoreMesh(**_MESH),
    scratch_types=[
        pltpu.VMEM((CH,), jnp.int32),
        pltpu.VMEM((CH,), jnp.float32),
        pltpu.VMEM((RPT,), jnp.float32),
        pltpu.VMEM_SHARED((NP,), jnp.float32),
    ],
)
def _deg_kernel(dst_hbm, out_hbm, dst_v, ones_v, zeros_v, acc):
    c = lax.axis_index("c")
    s = lax.axis_index("s")
    wid = s * NC + c
    one16 = jnp.ones((16,), jnp.float32)
    zero16 = jnp.zeros((16,), jnp.float32)

    def fill_ones(i, carry):
        ones_v[pl.ds(i * 16, 16)] = one16
        return carry

    lax.fori_loop(0, CH // 16, fill_ones, 0)

    def fill_zeros(i, carry):
        zeros_v[pl.ds(i * 16, 16)] = zero16
        return carry

    lax.fori_loop(0, RPT // 16, fill_zeros, 0)
    pltpu.sync_copy(zeros_v, acc.at[pl.ds(s * RPT, RPT)])
    plsc.subcore_barrier()

    ebase = wid * EPW

    def body(j, carry):
        pltpu.sync_copy(dst_hbm.at[pl.ds(ebase + j * CH, CH)], dst_v)
        pltpu.sync_copy(ones_v, acc.at[dst_v], add=True)
        return carry

    lax.fori_loop(0, NITER, body, 0)
    plsc.subcore_barrier()
    pltpu.sync_copy(acc.at[pl.ds(s * RPT, RPT)],
                    out_hbm.at[c, pl.ds(s * RPT, RPT)])


def _make_agg(F):
    @functools.partial(
        pl.kernel,
        out_type=jax.ShapeDtypeStruct((NC, NP, F), jnp.float32),
        mesh=plsc.VectorSubcoreMesh(**_MESH),
        scratch_types=[
            pltpu.VMEM((CH,), jnp.int32),
            pltpu.VMEM((CH,), jnp.int32),
            pltpu.VMEM((CH, F), jnp.float32),
            pltpu.VMEM_SHARED((NP, F), jnp.float32),
        ],
    )
    def agg(p_hbm, src_hbm, dst_hbm, out_hbm, src_v, dst_v, rows_v, acc):
        c = lax.axis_index("c")
        s = lax.axis_index("s")
        wid = s * NC + c
        zero16 = jnp.zeros((16,), jnp.float32)

        def zrow(i, carry):
            for k2 in range(F // 16):
                rows_v[i, pl.ds(k2 * 16, 16)] = zero16
            return carry

        lax.fori_loop(0, CH, zrow, 0)
        for k2 in range(RPT // CH):
            pltpu.sync_copy(rows_v, acc.at[pl.ds(s * RPT + k2 * CH, CH)])
        plsc.subcore_barrier()

        ebase = wid * EPW

        def body(j, carry):
            b = ebase + j * CH
            pltpu.sync_copy(src_hbm.at[pl.ds(b, CH)], src_v)
            pltpu.sync_copy(dst_hbm.at[pl.ds(b, CH)], dst_v)
            pltpu.sync_copy(p_hbm.at[src_v], rows_v)
            pltpu.sync_copy(rows_v, acc.at[dst_v], add=True)
            return carry

        lax.fori_loop(0, NITER, body, 0)
        plsc.subcore_barrier()
        pltpu.sync_copy(acc.at[pl.ds(s * RPT, RPT)],
                        out_hbm.at[c, pl.ds(s * RPT, RPT)])

    return agg


_agg = _make_agg(128)


def _mm1_body(x_ref, w_ref, dmat_ref, o_ref):
    o_ref[:, :64] = (jnp.dot(x_ref[...], w_ref[...],
                             preferred_element_type=jnp.float32)
                     * dmat_ref[:, :64])
    o_ref[:, 64:] = jnp.zeros((R, 64), jnp.float32)


def _mid_body(s_ref, p1_ref, dmat_ref, b1_ref, w2_ref, o_ref):
    d64 = dmat_ref[:, :64]
    aggv = s_ref[0, :, :64] + s_ref[1, :, :64] + p1_ref[:, :64]
    h = jnp.maximum(aggv * d64 + b1_ref[...], 0.0)
    o_ref[...] = (jnp.dot(h, w2_ref[...], preferred_element_type=jnp.float32)
                  * dmat_ref[...])


def _out_body(s_ref, p2_ref, dmat_ref, b2_ref, o_ref):
    o_ref[...] = ((s_ref[0] + s_ref[1] + p2_ref[...]) * dmat_ref[...]
                  + b2_ref[...])


def _mm1(x, W1, dmat):
    return pl.pallas_call(
        _mm1_body,
        grid=(GRID,),
        in_specs=[
            pl.BlockSpec((R, 128), lambda j: (j, 0)),
            pl.BlockSpec((128, 64), lambda j: (0, 0)),
            pl.BlockSpec((R, 128), lambda j: (j, 0)),
        ],
        out_specs=pl.BlockSpec((R, 128), lambda j: (j, 0)),
        out_shape=jax.ShapeDtypeStruct((N, 128), jnp.float32),
    )(x, W1, dmat)


def _mid(s1, p1, dmat, b1, W2):
    return pl.pallas_call(
        _mid_body,
        grid=(GRID,),
        in_specs=[
            pl.BlockSpec((NC, R, 128), lambda j: (0, j, 0)),
            pl.BlockSpec((R, 128), lambda j: (j, 0)),
            pl.BlockSpec((R, 128), lambda j: (j, 0)),
            pl.BlockSpec((1, 64), lambda j: (0, 0)),
            pl.BlockSpec((64, 128), lambda j: (0, 0)),
        ],
        out_specs=pl.BlockSpec((R, 128), lambda j: (j, 0)),
        out_shape=jax.ShapeDtypeStruct((N, 128), jnp.float32),
    )(s1, p1, dmat, b1, W2)


def _outk(s2, p2, dmat, b2):
    return pl.pallas_call(
        _out_body,
        grid=(GRID,),
        in_specs=[
            pl.BlockSpec((NC, R, 128), lambda j: (0, j, 0)),
            pl.BlockSpec((R, 128), lambda j: (j, 0)),
            pl.BlockSpec((R, 128), lambda j: (j, 0)),
            pl.BlockSpec((1, 128), lambda j: (0, 0)),
        ],
        out_specs=pl.BlockSpec((R, 128), lambda j: (j, 0)),
        out_shape=jax.ShapeDtypeStruct((N, 128), jnp.float32),
    )(s2, p2, dmat, b2)


def kernel(x, edge_index, W1, b1, W2, b2):
    src = edge_index[0].astype(jnp.int32)
    dst = edge_index[1].astype(jnp.int32)
    deg_parts = _deg_kernel(dst)
    deg = deg_parts[0] + deg_parts[1] + 1.0
    dinv = lax.rsqrt(deg)
    dmat = jnp.broadcast_to(dinv[:, None], (NP, 128))
    p1 = _mm1(x, W1, dmat)
    s1 = _agg(p1, src, dst)
    p2 = _mid(s1, p1, dmat, b1.reshape(1, 64), W2)
    s2 = _agg(p2, src, dst)
    return _outk(s2, p2, dmat, b2.reshape(1, 128))
